# baseline (speedup 1.0000x reference)
import functools

import jax
import jax.numpy as jnp
from jax import lax
from jax.experimental import pallas as pl
from jax.experimental.pallas import tpu as pltpu
from jax.experimental.pallas import tpu_sc as plsc

N = 10000
E = 320000
D_IN = 128
D_HID = 512
D_OUT = 200

NC = 2
NS = 16
CH = 128
CHUNKS = 158
CAPW = CHUNKS * CH
WIN = 5120
TRASH = WIN
W_PAD = 6144
RPW = W_PAD // NS

_MESH = plsc.VectorSubcoreMesh(
    core_axis_name="c", subcore_axis_name="s", num_cores=NC, num_subcores=NS
)

import dataclasses as _dataclasses

_SC_PARAMS = pltpu.CompilerParams()
if "needs_layout_passes" in pltpu.CompilerParams.__dataclass_fields__:
    _SC_PARAMS = _dataclasses.replace(_SC_PARAMS, needs_layout_passes=False)


def _nchunks(nch_hbm, nch_v, c):
    pltpu.sync_copy(nch_hbm.at[c], nch_v)
    return lax.reduce_max(nch_v[...], (0,))


def _deg_body(dst_hbm, zeros_hbm, nch_hbm, out_hbm, idx_v, ones_v, nch_v,
              acc_sh, sem):
    c = lax.axis_index("c")
    s = lax.axis_index("s")
    n2 = _nchunks(nch_hbm, nch_v, c)
    for i in range(CH // 16):
        ones_v[pl.ds(i * 16, 16)] = jnp.full((16,), 1.0, jnp.float32)
    pltpu.sync_copy(zeros_hbm.at[pl.ds(s * RPW, RPW)], acc_sh.at[pl.ds(s * RPW, RPW)])
    pltpu.sync_copy(dst_hbm.at[c].at[s], idx_v)
    plsc.subcore_barrier()

    @pl.loop(0, n2)
    def _issue(j):
        pltpu.async_copy(ones_v, acc_sh.at[idx_v.at[j]], sem, add=True)

    @pl.loop(0, n2)
    def _drain(j):
        pltpu.make_async_copy(ones_v, acc_sh.at[idx_v.at[0]], sem).wait()

    plsc.subcore_barrier()
    pltpu.sync_copy(acc_sh.at[pl.ds(s * RPW, RPW)], out_hbm.at[c].at[pl.ds(s * RPW, RPW)])


@jax.jit
def _sc_degree(dst_idx, zeros1, nch):
    k = pl.kernel(
        _deg_body,
        out_type=jax.ShapeDtypeStruct((NC, W_PAD), jnp.float32),
        mesh=_MESH,
        compiler_params=_SC_PARAMS,
        scratch_types=[
            pltpu.VMEM((CHUNKS, CH), jnp.int32),
            pltpu.VMEM((CH,), jnp.float32),
            pltpu.VMEM((16,), jnp.int32),
            pltpu.VMEM_SHARED((W_PAD,), jnp.float32),
            pltpu.SemaphoreType.DMA,
        ],
    )
    return k(dst_idx, zeros1, nch)


def _agg_body(G, x0_hbm, x1_hbm, src_hbm, dst_hbm, zeros_hbm, nch_hbm, out_hbm,
              srcv, dstv, nch_v, bufs, acc_sh, gsems, ssems):
    c = lax.axis_index("c")
    s = lax.axis_index("s")
    rows_mine = pl.ds(s * RPW, RPW)
    n2 = _nchunks(nch_hbm, nch_v, c)
    pltpu.sync_copy(src_hbm.at[c].at[s], srcv)
    pltpu.sync_copy(dst_hbm.at[c].at[s], dstv)
    for g in range(G):
        x_hbm = (x0_hbm, x1_hbm)[g]
        pltpu.sync_copy(zeros_hbm.at[rows_mine], acc_sh.at[rows_mine])
        plsc.subcore_barrier()

        def gather(j, k):
            pltpu.async_copy(x_hbm.at[srcv.at[j]], bufs[k], gsems[k])

        def scatter(j, k):
            pltpu.async_copy(bufs[k], acc_sh.at[dstv.at[j]], ssems[k], add=True)

        def wait_gather(k):
            pltpu.make_async_copy(x_hbm.at[srcv.at[0]], bufs[k], gsems[k]).wait()

        def wait_scatter(k):
            pltpu.make_async_copy(bufs[k], acc_sh.at[dstv.at[0]], ssems[k]).wait()

        gather(0, 0)

        @pl.loop(0, n2, step=2)
        def _pair(j):
            @pl.when(j > 0)
            def _():
                wait_scatter(1)
            gather(j + 1, 1)
            wait_gather(0)
            scatter(j, 0)
            wait_gather(1)
            scatter(j + 1, 1)
            wait_scatter(0)

            @pl.when(j + 2 < n2)
            def _():
                gather(j + 2, 0)

        wait_scatter(1)
        plsc.subcore_barrier()
        pltpu.sync_copy(acc_sh.at[rows_mine], out_hbm.at[c].at[g].at[rows_mine])
        plsc.subcore_barrier()


@functools.partial(jax.jit, static_argnums=0)
def _sc_aggregate(G, x0, x1, src_idx, dst_idx, zeros2, nch):
    k = pl.kernel(
        functools.partial(_agg_body, G),
        out_type=jax.ShapeDtypeStruct((NC, G, W_PAD, D_IN), jnp.float32),
        mesh=_MESH,
        compiler_params=_SC_PARAMS,
        scratch_types=[
            pltpu.VMEM((CHUNKS, CH), jnp.int32),
            pltpu.VMEM((CHUNKS, CH), jnp.int32),
            pltpu.VMEM((16,), jnp.int32),
            [pltpu.VMEM((CH, D_IN), jnp.float32) for _ in range(2)],
            pltpu.VMEM_SHARED((W_PAD, D_IN), jnp.float32),
            [pltpu.SemaphoreType.DMA for _ in range(2)],
            [pltpu.SemaphoreType.DMA for _ in range(2)],
        ],
    )
    return k(x0, x1, src_idx, dst_idx, zeros2, nch)


def _tc_body(raw0_ref, xs_ref, dis_ref, w1_ref, b1_ref, w2_ref, o_ref):
    dis = dis_ref[...]
    r = (raw0_ref[...] + xs_ref[...]) * dis
    h = jax.lax.dot(r, w1_ref[...], precision=jax.lax.Precision.HIGHEST)
    h = jnp.maximum(h + b1_ref[...], 0.0)
    t = jax.lax.dot(h, w2_ref[...], precision=jax.lax.Precision.HIGHEST)
    o_ref[...] = t * dis


BM = 1000


@jax.jit
def _tc_stage(raw0, xs, dis2, w1, b1r, w2):
    return pl.pallas_call(
        _tc_body,
        grid=(N // BM,),
        in_specs=[
            pl.BlockSpec((BM, D_IN), lambda i: (i, 0)),
            pl.BlockSpec((BM, D_IN), lambda i: (i, 0)),
            pl.BlockSpec((BM, 1), lambda i: (i, 0)),
            pl.BlockSpec((D_IN, D_HID), lambda i: (0, 0)),
            pl.BlockSpec((1, D_HID), lambda i: (0, 0)),
            pl.BlockSpec((D_HID, D_OUT), lambda i: (0, 0)),
        ],
        out_specs=pl.BlockSpec((BM, D_OUT), lambda i: (i, 0)),
        out_shape=jax.ShapeDtypeStruct((N, D_OUT), jnp.float32),
    )(raw0, xs, dis2, w1, b1r, w2)


def kernel(edges, node_features, W1, b1, W2, b2):
    src = edges[0]
    dst = edges[1]
    win = dst // WIN
    dstl = dst - win * WIN
    m0 = win == 0
    p0 = jnp.cumsum(m0) - 1
    p1 = jnp.cumsum(~m0) - 1
    n0 = p0[-1] + 1
    n1 = E - n0
    L0 = jnp.maximum(-(-n0 // NS), 1)
    L1 = jnp.maximum(-(-n1 // NS), 1)
    w0 = p0 // L0
    w1 = p1 // L1
    slot = jnp.where(m0, w0 * CAPW + (p0 - w0 * L0),
                     NS * CAPW + w1 * CAPW + (p1 - w1 * L1))
    srcp = jnp.zeros((NC * NS * CAPW,), jnp.int32).at[slot].set(
        src, mode="drop", unique_indices=True).reshape(NC, NS, CHUNKS, CH)
    dstp = jnp.full((NC * NS * CAPW,), TRASH, jnp.int32).at[slot].set(
        dstl, mode="drop", unique_indices=True).reshape(NC, NS, CHUNKS, CH)
    nch = jnp.stack([L0, L1])
    nch = jnp.clip((-(-nch // CH) + 1) // 2 * 2, 2, CHUNKS)
    nch16 = jnp.broadcast_to(nch[:, None], (NC, 16)).astype(jnp.int32)

    zeros1 = jnp.zeros((W_PAD,), jnp.float32)
    zeros2 = jnp.zeros((W_PAD, D_IN), jnp.float32)

    degp = _sc_degree(dstp, zeros1, nch16)
    deg = jnp.concatenate([degp[0, :WIN], degp[1, :WIN]])[:N] + 1.0
    dis2 = lax.rsqrt(deg)[:, None]

    xs = node_features * dis2
    raw1 = _sc_aggregate(1, xs, xs, srcp, dstp, zeros2, nch16)
    raw1 = raw1[:, 0, :WIN].reshape(NC * WIN, D_IN)[:N]
    ts = _tc_stage(raw1, xs, dis2, W1, b1[None, :], W2)
    tsp = jnp.pad(ts, ((0, 0), (0, 2 * D_IN - D_OUT)))
    raw2 = _sc_aggregate(2, tsp[:, :D_IN], tsp[:, D_IN:].copy(), srcp, dstp,
                         zeros2, nch16)
    raw2 = raw2[:, :, :WIN].transpose(0, 2, 1, 3).reshape(NC * WIN, 2 * D_IN)
    raw2 = raw2[:N, :D_OUT]
    return dis2 * (raw2 + ts) + b2[None, :]


# speedup vs baseline: 2.8898x; 2.8898x over previous
import functools

import jax
import jax.numpy as jnp
from jax import lax
from jax.experimental import pallas as pl
from jax.experimental.pallas import tpu as pltpu
from jax.experimental.pallas import tpu_sc as plsc

N = 10000
E = 320000
D_IN = 128
D_HID = 512
D_OUT = 200

NC = 2
NS = 16
CH = 128
CHUNKS = 158
CAPW = CHUNKS * CH
EPS = E // NS
WIN = 5120
TRASH = WIN
W_PAD = 6144
RPW = W_PAD // NS

_MESH = plsc.VectorSubcoreMesh(
    core_axis_name="c", subcore_axis_name="s", num_cores=NC, num_subcores=NS
)

import dataclasses as _dataclasses

_SC_PARAMS = pltpu.CompilerParams()
if "needs_layout_passes" in pltpu.CompilerParams.__dataclass_fields__:
    _SC_PARAMS = _dataclasses.replace(_SC_PARAMS, needs_layout_passes=False)


def _nchunks(nch_hbm, nch_v, c, s):
    pltpu.sync_copy(nch_hbm.at[c].at[s], nch_v)
    return lax.reduce_max(nch_v[...], (0,))


def _deg_body(dst_hbm, zeros_hbm, nch_hbm, out_hbm, idx_v, ones_v, nch_v,
              acc_sh, sem):
    c = lax.axis_index("c")
    s = lax.axis_index("s")
    n2 = _nchunks(nch_hbm, nch_v, c, s)
    for i in range(CH // 16):
        ones_v[pl.ds(i * 16, 16)] = jnp.full((16,), 1.0, jnp.float32)
    pltpu.sync_copy(zeros_hbm.at[pl.ds(s * RPW, RPW)], acc_sh.at[pl.ds(s * RPW, RPW)])
    pltpu.sync_copy(dst_hbm.at[s].at[c], idx_v)
    plsc.subcore_barrier()

    @pl.loop(0, n2)
    def _issue(j):
        pltpu.async_copy(ones_v, acc_sh.at[idx_v.at[j]], sem, add=True)

    @pl.loop(0, n2)
    def _drain(j):
        pltpu.make_async_copy(ones_v, acc_sh.at[idx_v.at[0]], sem).wait()

    plsc.subcore_barrier()
    pltpu.sync_copy(acc_sh.at[pl.ds(s * RPW, RPW)], out_hbm.at[c].at[pl.ds(s * RPW, RPW)])


@jax.jit
def _sc_degree(dst_idx, zeros1, nch):
    k = pl.kernel(
        _deg_body,
        out_type=jax.ShapeDtypeStruct((NC, W_PAD), jnp.float32),
        mesh=_MESH,
        compiler_params=_SC_PARAMS,
        scratch_types=[
            pltpu.VMEM((CHUNKS, CH), jnp.int32),
            pltpu.VMEM((CH,), jnp.float32),
            pltpu.VMEM((16,), jnp.int32),
            pltpu.VMEM_SHARED((W_PAD,), jnp.float32),
            pltpu.SemaphoreType.DMA,
        ],
    )
    return k(dst_idx, zeros1, nch)


def _agg_body(G, x0_hbm, x1_hbm, src_hbm, dst_hbm, zeros_hbm, nch_hbm, out_hbm,
              srcv, dstv, nch_v, bufs, acc_sh, gsems, ssems):
    c = lax.axis_index("c")
    s = lax.axis_index("s")
    rows_mine = pl.ds(s * RPW, RPW)
    n2 = _nchunks(nch_hbm, nch_v, c, s)
    pltpu.sync_copy(src_hbm.at[s].at[c], srcv)
    pltpu.sync_copy(dst_hbm.at[s].at[c], dstv)
    for g in range(G):
        x_hbm = (x0_hbm, x1_hbm)[g]
        pltpu.sync_copy(zeros_hbm.at[rows_mine], acc_sh.at[rows_mine])
        plsc.subcore_barrier()

        def gather(j, k):
            pltpu.async_copy(x_hbm.at[srcv.at[j]], bufs[k], gsems[k])

        def scatter(j, k):
            pltpu.async_copy(bufs[k], acc_sh.at[dstv.at[j]], ssems[k], add=True)

        def wait_gather(k):
            pltpu.make_async_copy(x_hbm.at[srcv.at[0]], bufs[k], gsems[k]).wait()

        def wait_scatter(k):
            pltpu.make_async_copy(bufs[k], acc_sh.at[dstv.at[0]], ssems[k]).wait()

        gather(0, 0)

        @pl.loop(0, n2, step=2)
        def _pair(j):
            @pl.when(j > 0)
            def _():
                wait_scatter(1)
            gather(j + 1, 1)
            wait_gather(0)
            scatter(j, 0)
            wait_gather(1)
            scatter(j + 1, 1)
            wait_scatter(0)

            @pl.when(j + 2 < n2)
            def _():
                gather(j + 2, 0)

        wait_scatter(1)
        plsc.subcore_barrier()
        pltpu.sync_copy(acc_sh.at[rows_mine], out_hbm.at[c].at[g].at[rows_mine])
        plsc.subcore_barrier()


@functools.partial(jax.jit, static_argnums=0)
def _sc_aggregate(G, x0, x1, src_idx, dst_idx, zeros2, nch):
    k = pl.kernel(
        functools.partial(_agg_body, G),
        out_type=jax.ShapeDtypeStruct((NC, G, W_PAD, D_IN), jnp.float32),
        mesh=_MESH,
        compiler_params=_SC_PARAMS,
        scratch_types=[
            pltpu.VMEM((CHUNKS, CH), jnp.int32),
            pltpu.VMEM((CHUNKS, CH), jnp.int32),
            pltpu.VMEM((16,), jnp.int32),
            [pltpu.VMEM((CH, D_IN), jnp.float32) for _ in range(2)],
            pltpu.VMEM_SHARED((W_PAD, D_IN), jnp.float32),
            [pltpu.SemaphoreType.DMA for _ in range(2)],
            [pltpu.SemaphoreType.DMA for _ in range(2)],
        ],
    )
    return k(x0, x1, src_idx, dst_idx, zeros2, nch)


def _tc_body(raw0_ref, xs_ref, dis_ref, w1_ref, b1_ref, w2_ref, o_ref):
    dis = dis_ref[...]
    r = (raw0_ref[...] + xs_ref[...]) * dis
    h = jax.lax.dot(r, w1_ref[...], precision=jax.lax.Precision.HIGHEST)
    h = jnp.maximum(h + b1_ref[...], 0.0)
    t = jax.lax.dot(h, w2_ref[...], precision=jax.lax.Precision.HIGHEST)
    o_ref[...] = t * dis


BM = 1000


@jax.jit
def _tc_stage(raw0, xs, dis2, w1, b1r, w2):
    return pl.pallas_call(
        _tc_body,
        grid=(N // BM,),
        in_specs=[
            pl.BlockSpec((BM, D_IN), lambda i: (i, 0)),
            pl.BlockSpec((BM, D_IN), lambda i: (i, 0)),
            pl.BlockSpec((BM, 1), lambda i: (i, 0)),
            pl.BlockSpec((D_IN, D_HID), lambda i: (0, 0)),
            pl.BlockSpec((1, D_HID), lambda i: (0, 0)),
            pl.BlockSpec((D_HID, D_OUT), lambda i: (0, 0)),
        ],
        out_specs=pl.BlockSpec((BM, D_OUT), lambda i: (i, 0)),
        out_shape=jax.ShapeDtypeStruct((N, D_OUT), jnp.float32),
    )(raw0, xs, dis2, w1, b1r, w2)


def kernel(edges, node_features, W1, b1, W2, b2):
    src = edges[0].reshape(NS, EPS)
    dst = edges[1].reshape(NS, EPS)
    win = dst // WIN
    dstl = dst - win * WIN
    m0 = win == 0
    r0 = jnp.cumsum(m0, axis=1)
    pos_in = jnp.arange(1, EPS + 1, dtype=jnp.int32)[None, :]
    base = (jnp.arange(NS, dtype=jnp.int32) * (NC * CAPW))[:, None]
    slot = base + jnp.where(m0, r0 - 1, CAPW + (pos_in - r0) - 1)
    srcp = jnp.zeros((NS * NC * CAPW,), jnp.int32).at[slot.reshape(-1)].add(
        src.reshape(-1), mode="drop", unique_indices=True)
    dstp = jnp.zeros((NS * NC * CAPW,), jnp.int32).at[slot.reshape(-1)].add(
        dstl.reshape(-1) + 1, mode="drop", unique_indices=True)
    dstp = jnp.where(dstp == 0, TRASH, dstp - 1)
    srcp = srcp.reshape(NS, NC, CHUNKS, CH)
    dstp = dstp.reshape(NS, NC, CHUNKS, CH)
    n0w = r0[:, -1]
    nch = jnp.stack([n0w, EPS - n0w])
    nch = jnp.clip((-(-nch // CH) + 1) // 2 * 2, 2, CHUNKS)
    nch16 = jnp.broadcast_to(nch[:, :, None], (NC, NS, 16)).astype(jnp.int32)

    zeros1 = jnp.zeros((W_PAD,), jnp.float32)
    zeros2 = jnp.zeros((W_PAD, D_IN), jnp.float32)

    degp = _sc_degree(dstp, zeros1, nch16)
    deg = jnp.concatenate([degp[0, :WIN], degp[1, :WIN]])[:N] + 1.0
    dis2 = lax.rsqrt(deg)[:, None]

    xs = node_features * dis2
    raw1 = _sc_aggregate(1, xs, xs, srcp, dstp, zeros2, nch16)
    raw1 = raw1[:, 0, :WIN].reshape(NC * WIN, D_IN)[:N]
    ts = _tc_stage(raw1, xs, dis2, W1, b1[None, :], W2)
    tsp = jnp.pad(ts, ((0, 0), (0, 2 * D_IN - D_OUT)))
    raw2 = _sc_aggregate(2, tsp[:, :D_IN], tsp[:, D_IN:].copy(), srcp, dstp,
                         zeros2, nch16)
    raw2 = raw2[:, :, :WIN].transpose(0, 2, 1, 3).reshape(NC * WIN, 2 * D_IN)
    raw2 = raw2[:N, :D_OUT]
    return dis2 * (raw2 + ts) + b2[None, :]


# speedup vs baseline: 2.9033x; 1.0047x over previous
import functools

import jax
import jax.numpy as jnp
from jax import lax
from jax.experimental import pallas as pl
from jax.experimental.pallas import tpu as pltpu
from jax.experimental.pallas import tpu_sc as plsc

N = 10000
E = 320000
D_IN = 128
D_HID = 512
D_OUT = 200

NC = 2
NS = 16
CH = 128
CHUNKS = 158
CAPW = CHUNKS * CH
EPS = E // NS
WIN = 5120
TRASH = WIN
W_PAD = 6144
RPW = W_PAD // NS

_MESH = plsc.VectorSubcoreMesh(
    core_axis_name="c", subcore_axis_name="s", num_cores=NC, num_subcores=NS
)

import dataclasses as _dataclasses

_SC_PARAMS = pltpu.CompilerParams()
if "needs_layout_passes" in pltpu.CompilerParams.__dataclass_fields__:
    _SC_PARAMS = _dataclasses.replace(_SC_PARAMS, needs_layout_passes=False)


def _nchunks(nch_hbm, nch_v, c, s):
    pltpu.sync_copy(nch_hbm.at[c].at[s], nch_v)
    return lax.reduce_max(nch_v[...], (0,))


def _deg_body(dst_hbm, zeros_hbm, nch_hbm, out_hbm, idx_v, ones_v, nch_v,
              acc_sh, sem):
    c = lax.axis_index("c")
    s = lax.axis_index("s")
    n2 = _nchunks(nch_hbm, nch_v, c, s)
    for i in range(CH // 16):
        ones_v[pl.ds(i * 16, 16)] = jnp.full((16,), 1.0, jnp.float32)
    pltpu.sync_copy(zeros_hbm.at[pl.ds(s * RPW, RPW)], acc_sh.at[pl.ds(s * RPW, RPW)])
    pltpu.sync_copy(dst_hbm.at[s].at[c], idx_v)
    plsc.subcore_barrier()

    @pl.loop(0, n2)
    def _issue(j):
        pltpu.async_copy(ones_v, acc_sh.at[idx_v.at[j]], sem, add=True)

    @pl.loop(0, n2)
    def _drain(j):
        pltpu.make_async_copy(ones_v, acc_sh.at[idx_v.at[0]], sem).wait()

    plsc.subcore_barrier()
    pltpu.sync_copy(acc_sh.at[pl.ds(s * RPW, RPW)], out_hbm.at[c].at[pl.ds(s * RPW, RPW)])


@jax.jit
def _sc_degree(dst_idx, zeros1, nch):
    k = pl.kernel(
        _deg_body,
        out_type=jax.ShapeDtypeStruct((NC, W_PAD), jnp.float32),
        mesh=_MESH,
        compiler_params=_SC_PARAMS,
        scratch_types=[
            pltpu.VMEM((CHUNKS, CH), jnp.int32),
            pltpu.VMEM((CH,), jnp.float32),
            pltpu.VMEM((16,), jnp.int32),
            pltpu.VMEM_SHARED((W_PAD,), jnp.float32),
            pltpu.SemaphoreType.DMA,
        ],
    )
    return k(dst_idx, zeros1, nch)


def _agg_body(G, x0_hbm, x1_hbm, src_hbm, dst_hbm, zeros_hbm, nch_hbm, out_hbm,
              srcv, dstv, nch_v, bufs, acc_sh, gsems, ssems):
    c = lax.axis_index("c")
    s = lax.axis_index("s")
    rows_mine = pl.ds(s * RPW, RPW)
    n2 = _nchunks(nch_hbm, nch_v, c, s)
    pltpu.sync_copy(src_hbm.at[s].at[c], srcv)
    pltpu.sync_copy(dst_hbm.at[s].at[c], dstv)
    for g in range(G):
        x_hbm = (x0_hbm, x1_hbm)[g]
        pltpu.sync_copy(zeros_hbm.at[rows_mine], acc_sh.at[rows_mine])
        plsc.subcore_barrier()

        def gather(j, k):
            pltpu.async_copy(x_hbm.at[srcv.at[j]], bufs[k], gsems[k])

        def scatter(j, k):
            pltpu.async_copy(bufs[k], acc_sh.at[dstv.at[j]], ssems[k], add=True)

        def wait_gather(k):
            pltpu.make_async_copy(x_hbm.at[srcv.at[0]], bufs[k], gsems[k]).wait()

        def wait_scatter(k):
            pltpu.make_async_copy(bufs[k], acc_sh.at[dstv.at[0]], ssems[k]).wait()

        gather(0, 0)

        @pl.loop(0, n2, step=2)
        def _pair(j):
            @pl.when(j > 0)
            def _():
                wait_scatter(1)
            gather(j + 1, 1)
            wait_gather(0)
            scatter(j, 0)
            wait_gather(1)
            scatter(j + 1, 1)
            wait_scatter(0)

            @pl.when(j + 2 < n2)
            def _():
                gather(j + 2, 0)

        wait_scatter(1)
        plsc.subcore_barrier()
        pltpu.sync_copy(acc_sh.at[rows_mine], out_hbm.at[c].at[g].at[rows_mine])
        plsc.subcore_barrier()


@functools.partial(jax.jit, static_argnums=0)
def _sc_aggregate(G, x0, x1, src_idx, dst_idx, zeros2, nch):
    k = pl.kernel(
        functools.partial(_agg_body, G),
        out_type=jax.ShapeDtypeStruct((NC, G, W_PAD, D_IN), jnp.float32),
        mesh=_MESH,
        compiler_params=_SC_PARAMS,
        scratch_types=[
            pltpu.VMEM((CHUNKS, CH), jnp.int32),
            pltpu.VMEM((CHUNKS, CH), jnp.int32),
            pltpu.VMEM((16,), jnp.int32),
            [pltpu.VMEM((CH, D_IN), jnp.float32) for _ in range(2)],
            pltpu.VMEM_SHARED((W_PAD, D_IN), jnp.float32),
            [pltpu.SemaphoreType.DMA for _ in range(2)],
            [pltpu.SemaphoreType.DMA for _ in range(2)],
        ],
    )
    return k(x0, x1, src_idx, dst_idx, zeros2, nch)


def _tc_body(raw0_ref, xs_ref, dis_ref, w1_ref, b1_ref, w2_ref,
             o_ref, o0_ref, o1_ref):
    dis = dis_ref[...]
    r = (raw0_ref[...] + xs_ref[...]) * dis
    h = jax.lax.dot(r, w1_ref[...], precision=jax.lax.Precision.HIGHEST)
    h = jnp.maximum(h + b1_ref[...], 0.0)
    t = jax.lax.dot(h, w2_ref[...], precision=jax.lax.Precision.HIGHEST)
    ts = t * dis
    o_ref[...] = ts[:, :D_OUT]
    o0_ref[...] = ts[:, :D_IN]
    o1_ref[...] = ts[:, D_IN:]


BM = 1000


@jax.jit
def _tc_stage(raw0, xs, dis2, w1, b1r, w2p):
    return pl.pallas_call(
        _tc_body,
        grid=(N // BM,),
        in_specs=[
            pl.BlockSpec((BM, D_IN), lambda i: (i, 0)),
            pl.BlockSpec((BM, D_IN), lambda i: (i, 0)),
            pl.BlockSpec((BM, 1), lambda i: (i, 0)),
            pl.BlockSpec((D_IN, D_HID), lambda i: (0, 0)),
            pl.BlockSpec((1, D_HID), lambda i: (0, 0)),
            pl.BlockSpec((D_HID, 2 * D_IN), lambda i: (0, 0)),
        ],
        out_specs=[
            pl.BlockSpec((BM, D_OUT), lambda i: (i, 0)),
            pl.BlockSpec((BM, D_IN), lambda i: (i, 0)),
            pl.BlockSpec((BM, D_IN), lambda i: (i, 0)),
        ],
        out_shape=[
            jax.ShapeDtypeStruct((N, D_OUT), jnp.float32),
            jax.ShapeDtypeStruct((N, D_IN), jnp.float32),
            jax.ShapeDtypeStruct((N, D_IN), jnp.float32),
        ],
    )(raw0, xs, dis2, w1, b1r, w2p)


def kernel(edges, node_features, W1, b1, W2, b2):
    src = edges[0].reshape(NS, EPS)
    dst = edges[1].reshape(NS, EPS)
    win = dst // WIN
    dstl = dst - win * WIN
    m0 = win == 0
    r0 = jnp.cumsum(m0, axis=1)
    pos_in = jnp.arange(1, EPS + 1, dtype=jnp.int32)[None, :]
    base = (jnp.arange(NS, dtype=jnp.int32) * (NC * CAPW))[:, None]
    slot = base + jnp.where(m0, r0 - 1, CAPW + (pos_in - r0) - 1)
    srcp = jnp.zeros((NS * NC * CAPW,), jnp.int32).at[slot.reshape(-1)].add(
        src.reshape(-1), mode="drop", unique_indices=True)
    dstp = jnp.zeros((NS * NC * CAPW,), jnp.int32).at[slot.reshape(-1)].add(
        dstl.reshape(-1) + 1, mode="drop", unique_indices=True)
    dstp = jnp.where(dstp == 0, TRASH, dstp - 1)
    srcp = srcp.reshape(NS, NC, CHUNKS, CH)
    dstp = dstp.reshape(NS, NC, CHUNKS, CH)
    n0w = r0[:, -1]
    nch = jnp.stack([n0w, EPS - n0w])
    nch = jnp.clip((-(-nch // CH) + 1) // 2 * 2, 2, CHUNKS)
    nch16 = jnp.broadcast_to(nch[:, :, None], (NC, NS, 16)).astype(jnp.int32)

    zeros1 = jnp.zeros((W_PAD,), jnp.float32)
    zeros2 = jnp.zeros((W_PAD, D_IN), jnp.float32)

    degp = _sc_degree(dstp, zeros1, nch16)
    deg = jnp.concatenate([degp[0, :WIN], degp[1, :WIN]])[:N] + 1.0
    dis2 = lax.rsqrt(deg)[:, None]

    xs = node_features * dis2
    raw1 = _sc_aggregate(1, xs, xs, srcp, dstp, zeros2, nch16)
    raw1 = raw1[:, 0, :WIN].reshape(NC * WIN, D_IN)[:N]
    w2p = jnp.pad(W2, ((0, 0), (0, 2 * D_IN - D_OUT)))
    ts, ts0, ts1 = _tc_stage(raw1, xs, dis2, W1, b1[None, :], w2p)
    raw2 = _sc_aggregate(2, ts0, ts1, srcp, dstp, zeros2, nch16)
    raw2 = raw2[:, :, :WIN].transpose(0, 2, 1, 3).reshape(NC * WIN, 2 * D_IN)
    raw2 = raw2[:N, :D_OUT]
    return dis2 * (raw2 + ts) + b2[None, :]


# speedup vs baseline: 2.9083x; 1.0017x over previous
import functools

import jax
import jax.numpy as jnp
from jax import lax
from jax.experimental import pallas as pl
from jax.experimental.pallas import tpu as pltpu
from jax.experimental.pallas import tpu_sc as plsc

N = 10000
E = 320000
D_IN = 128
D_HID = 512
D_OUT = 200

NC = 2
NS = 16
CH = 128
CHUNKS = 158
CAPW = CHUNKS * CH
EPS = E // NS
WIN = 5120
TRASH = WIN
W_PAD = 6144
RPW = W_PAD // NS

@functools.cache
def _mesh():
    return plsc.VectorSubcoreMesh(
        core_axis_name="c", subcore_axis_name="s", num_cores=NC, num_subcores=NS
    )


import dataclasses as _dataclasses

_SC_PARAMS = pltpu.CompilerParams()
if "needs_layout_passes" in pltpu.CompilerParams.__dataclass_fields__:
    _SC_PARAMS = _dataclasses.replace(_SC_PARAMS, needs_layout_passes=False)


def _nchunks(nch_hbm, nch_v, c, s):
    pltpu.sync_copy(nch_hbm.at[c].at[s], nch_v)
    return lax.reduce_max(nch_v[...], (0,))


def _deg_body(dst_hbm, zeros_hbm, nch_hbm, out_hbm, idx_v, ones_v, nch_v,
              acc_sh, sem):
    c = lax.axis_index("c")
    s = lax.axis_index("s")
    n2 = _nchunks(nch_hbm, nch_v, c, s)
    for i in range(CH // 16):
        ones_v[pl.ds(i * 16, 16)] = jnp.full((16,), 1.0, jnp.float32)
    pltpu.sync_copy(zeros_hbm.at[pl.ds(s * RPW, RPW)], acc_sh.at[pl.ds(s * RPW, RPW)])
    pltpu.sync_copy(dst_hbm.at[s].at[c], idx_v)
    plsc.subcore_barrier()

    @pl.loop(0, n2)
    def _issue(j):
        pltpu.async_copy(ones_v, acc_sh.at[idx_v.at[j]], sem, add=True)

    @pl.loop(0, n2)
    def _drain(j):
        pltpu.make_async_copy(ones_v, acc_sh.at[idx_v.at[0]], sem).wait()

    plsc.subcore_barrier()
    pltpu.sync_copy(acc_sh.at[pl.ds(s * RPW, RPW)], out_hbm.at[c].at[pl.ds(s * RPW, RPW)])


@jax.jit
def _sc_degree(dst_idx, zeros1, nch):
    k = pl.kernel(
        _deg_body,
        out_type=jax.ShapeDtypeStruct((NC, W_PAD), jnp.float32),
        mesh=_mesh(),
        compiler_params=_SC_PARAMS,
        scratch_types=[
            pltpu.VMEM((CHUNKS, CH), jnp.int32),
            pltpu.VMEM((CH,), jnp.float32),
            pltpu.VMEM((16,), jnp.int32),
            pltpu.VMEM_SHARED((W_PAD,), jnp.float32),
            pltpu.SemaphoreType.DMA,
        ],
    )
    return k(dst_idx, zeros1, nch)


def _agg_body(G, x0_hbm, x1_hbm, src_hbm, dst_hbm, zeros_hbm, nch_hbm, out_hbm,
              srcv, dstv, nch_v, bufs, acc_sh, gsems, ssems):
    c = lax.axis_index("c")
    s = lax.axis_index("s")
    rows_mine = pl.ds(s * RPW, RPW)
    n2 = _nchunks(nch_hbm, nch_v, c, s)
    pltpu.sync_copy(src_hbm.at[s].at[c], srcv)
    pltpu.sync_copy(dst_hbm.at[s].at[c], dstv)
    for g in range(G):
        x_hbm = (x0_hbm, x1_hbm)[g]
        pltpu.sync_copy(zeros_hbm.at[rows_mine], acc_sh.at[rows_mine])
        plsc.subcore_barrier()

        def gather(j, k):
            pltpu.async_copy(x_hbm.at[srcv.at[j]], bufs[k], gsems[k])

        def scatter(j, k):
            pltpu.async_copy(bufs[k], acc_sh.at[dstv.at[j]], ssems[k], add=True)

        def wait_gather(k):
            pltpu.make_async_copy(x_hbm.at[srcv.at[0]], bufs[k], gsems[k]).wait()

        def wait_scatter(k):
            pltpu.make_async_copy(bufs[k], acc_sh.at[dstv.at[0]], ssems[k]).wait()

        gather(0, 0)

        @pl.loop(0, n2, step=2)
        def _pair(j):
            @pl.when(j > 0)
            def _():
                wait_scatter(1)
            gather(j + 1, 1)
            wait_gather(0)
            scatter(j, 0)
            wait_gather(1)
            scatter(j + 1, 1)
            wait_scatter(0)

            @pl.when(j + 2 < n2)
            def _():
                gather(j + 2, 0)

        wait_scatter(1)
        plsc.subcore_barrier()
        pltpu.sync_copy(acc_sh.at[rows_mine], out_hbm.at[c].at[g].at[rows_mine])
        plsc.subcore_barrier()


@functools.partial(jax.jit, static_argnums=0)
def _sc_aggregate(G, x0, x1, src_idx, dst_idx, zeros2, nch):
    k = pl.kernel(
        functools.partial(_agg_body, G),
        out_type=jax.ShapeDtypeStruct((NC, G, W_PAD, D_IN), jnp.float32),
        mesh=_mesh(),
        compiler_params=_SC_PARAMS,
        scratch_types=[
            pltpu.VMEM((CHUNKS, CH), jnp.int32),
            pltpu.VMEM((CHUNKS, CH), jnp.int32),
            pltpu.VMEM((16,), jnp.int32),
            [pltpu.VMEM((CH, D_IN), jnp.float32) for _ in range(2)],
            pltpu.VMEM_SHARED((W_PAD, D_IN), jnp.float32),
            [pltpu.SemaphoreType.DMA for _ in range(2)],
            [pltpu.SemaphoreType.DMA for _ in range(2)],
        ],
    )
    return k(x0, x1, src_idx, dst_idx, zeros2, nch)


def _tc_body(raw0_ref, xs_ref, dis_ref, w1_ref, b1_ref, w2_ref,
             o_ref, o0_ref, o1_ref):
    dis = dis_ref[...]
    r = (raw0_ref[...] + xs_ref[...]) * dis
    h = jax.lax.dot(r, w1_ref[...], precision=jax.lax.Precision.HIGHEST)
    h = jnp.maximum(h + b1_ref[...], 0.0)
    t = jax.lax.dot(h, w2_ref[...], precision=jax.lax.Precision.HIGHEST)
    ts = t * dis
    o_ref[...] = ts[:, :D_OUT]
    o0_ref[...] = ts[:, :D_IN]
    o1_ref[...] = ts[:, D_IN:]


BM = 1000


@jax.jit
def _tc_stage(raw0, xs, dis2, w1, b1r, w2p):
    return pl.pallas_call(
        _tc_body,
        grid=(N // BM,),
        in_specs=[
            pl.BlockSpec((BM, D_IN), lambda i: (i, 0)),
            pl.BlockSpec((BM, D_IN), lambda i: (i, 0)),
            pl.BlockSpec((BM, 1), lambda i: (i, 0)),
            pl.BlockSpec((D_IN, D_HID), lambda i: (0, 0)),
            pl.BlockSpec((1, D_HID), lambda i: (0, 0)),
            pl.BlockSpec((D_HID, 2 * D_IN), lambda i: (0, 0)),
        ],
        out_specs=[
            pl.BlockSpec((BM, D_OUT), lambda i: (i, 0)),
            pl.BlockSpec((BM, D_IN), lambda i: (i, 0)),
            pl.BlockSpec((BM, D_IN), lambda i: (i, 0)),
        ],
        out_shape=[
            jax.ShapeDtypeStruct((N, D_OUT), jnp.float32),
            jax.ShapeDtypeStruct((N, D_IN), jnp.float32),
            jax.ShapeDtypeStruct((N, D_IN), jnp.float32),
        ],
    )(raw0, xs, dis2, w1, b1r, w2p)


def kernel(edges, node_features, W1, b1, W2, b2):
    src = edges[0].reshape(NS, EPS)
    dst = edges[1].reshape(NS, EPS)
    win = dst // WIN
    dstl = dst - win * WIN
    m0 = win == 0
    r0 = jnp.cumsum(m0, axis=1)
    pos_in = jnp.arange(1, EPS + 1, dtype=jnp.int32)[None, :]
    base = (jnp.arange(NS, dtype=jnp.int32) * (NC * CAPW))[:, None]
    slot = base + jnp.where(m0, r0 - 1, CAPW + (pos_in - r0) - 1)
    srcp = jnp.zeros((NS * NC * CAPW,), jnp.int32).at[slot.reshape(-1)].add(
        src.reshape(-1), mode="drop", unique_indices=True)
    dstp = jnp.zeros((NS * NC * CAPW,), jnp.int32).at[slot.reshape(-1)].add(
        dstl.reshape(-1) + 1, mode="drop", unique_indices=True)
    dstp = jnp.where(dstp == 0, TRASH, dstp - 1)
    srcp = srcp.reshape(NS, NC, CHUNKS, CH)
    dstp = dstp.reshape(NS, NC, CHUNKS, CH)
    n0w = r0[:, -1]
    nch = jnp.stack([n0w, EPS - n0w])
    nch = jnp.clip((-(-nch // CH) + 1) // 2 * 2, 2, CHUNKS)
    nch16 = jnp.broadcast_to(nch[:, :, None], (NC, NS, 16)).astype(jnp.int32)

    zeros1 = jnp.zeros((W_PAD,), jnp.float32)
    zeros2 = jnp.zeros((W_PAD, D_IN), jnp.float32)

    degp = _sc_degree(dstp, zeros1, nch16)
    deg = jnp.concatenate([degp[0, :WIN], degp[1, :WIN]])[:N] + 1.0
    dis2 = lax.rsqrt(deg)[:, None]

    xs = node_features * dis2
    raw1 = _sc_aggregate(1, xs, xs, srcp, dstp, zeros2, nch16)
    raw1 = raw1[:, 0, :WIN].reshape(NC * WIN, D_IN)[:N]
    w2p = jnp.pad(W2, ((0, 0), (0, 2 * D_IN - D_OUT)))
    ts, ts0, ts1 = _tc_stage(raw1, xs, dis2, W1, b1[None, :], w2p)
    raw2 = _sc_aggregate(2, ts0, ts1, srcp, dstp, zeros2, nch16)
    raw2 = raw2[:, :, :WIN].transpose(0, 2, 1, 3).reshape(NC * WIN, 2 * D_IN)
    raw2 = raw2[:N, :D_OUT]
    return dis2 * (raw2 + ts) + b2[None, :]
